# baseline (speedup 1.0000x reference)
import functools
import math

import jax
import jax.numpy as jnp
from jax import lax
from jax.experimental import pallas as pl
from jax.experimental.pallas import tpu as pltpu
from jax.experimental.pallas import tpu_sc as plsc

_LANES = 16


def _build_sc_embed(N, V, D, S, n_per_w, C, NC):
    mesh = plsc.VectorSubcoreMesh(core_axis_name="c", subcore_axis_name="s")
    scale = math.sqrt(float(D))
    vecs_per_row = D // _LANES
    n_chunks = n_per_w // C

    @functools.partial(
        pl.kernel,
        out_type=jax.ShapeDtypeStruct((N, D), jnp.float32),
        mesh=mesh,
        scratch_types=[
            pltpu.VMEM((n_per_w,), jnp.int32),
            pltpu.VMEM((C, D), jnp.float32),
            pltpu.VMEM((C, D), jnp.float32),
            pltpu.SemaphoreType.DMA,
        ],
    )
    def sc_embed(x_hbm, table_hbm, pe_hbm, out_hbm, idx_v, rows_v, pe_v, sem):
        wid = lax.axis_index("s") * NC + lax.axis_index("c")
        base = wid * n_per_w
        pos_base = base % S
        pltpu.sync_copy(x_hbm.at[pl.ds(base, n_per_w)], idx_v)

        def chunk_body(c, carry):
            row0 = base + c * C
            pos0 = pos_base + c * C
            pltpu.async_copy(
                table_hbm.at[idx_v.at[pl.ds(c * C, C)]], rows_v, sem
            ).wait()
            pltpu.sync_copy(pe_hbm.at[pl.ds(pos0, C)], pe_v)

            def vec_body(i, carry2):
                r = i // vecs_per_row
                j = i % vecs_per_row
                sl = pl.ds(j * _LANES, _LANES)
                rows_v[r, sl] = rows_v[r, sl] * scale + pe_v[r, sl]
                return carry2

            lax.fori_loop(0, C * vecs_per_row, vec_body, 0, unroll=4)
            pltpu.sync_copy(rows_v, out_hbm.at[pl.ds(row0, C)])
            return carry

        lax.fori_loop(0, n_chunks, chunk_body, 0)

    return sc_embed


@jax.jit
def kernel(x, table, pe):
    B, S = x.shape
    V, D = table.shape
    N = B * S
    info = plsc.get_sparse_core_info()
    NC, NS = info.num_cores, info.num_subcores
    NW = NC * NS
    n_per_w = N // NW
    C = 16

    x_flat = x.reshape(N).astype(jnp.int32)
    pe2 = pe.reshape(pe.shape[1], D)[:S]

    sc_embed = _build_sc_embed(N, V, D, S, n_per_w, C, NC)
    out = sc_embed(x_flat, table, pe2)
    return out.reshape(B, S, D)
